# baseline (speedup 1.0000x reference)
import functools

import jax
import jax.numpy as jnp
from jax import lax
from jax.experimental import pallas as pl
from jax.experimental.pallas import tpu as pltpu
from jax.experimental.pallas import tpu_sc as plsc

USER_NUM = 100000
ITEM_NUM = 1000000
E = 64
B = 4096
N = 1024

_info = plsc.get_sparse_core_info()
_NC, _NS = _info.num_cores, _info.num_subcores
_NW = _NC * _NS
_BPW = B // _NW
_NPW = N // _NW

_sc_mesh = plsc.VectorSubcoreMesh(core_axis_name="c", subcore_axis_name="s")


@functools.partial(
    pl.kernel,
    mesh=_sc_mesh,
    compiler_params=pltpu.CompilerParams(use_tc_tiling_on_sc=False),
    out_type=[
        jax.ShapeDtypeStruct((B, E), jnp.float32),
        jax.ShapeDtypeStruct((B, E), jnp.float32),
        jax.ShapeDtypeStruct((N, E), jnp.float32),
        jax.ShapeDtypeStruct((N, E), jnp.float32),
    ],
    scratch_types=[
        pltpu.VMEM((_BPW,), jnp.int32),
        pltpu.VMEM((_BPW,), jnp.int32),
        pltpu.VMEM((_NPW,), jnp.int32),
        pltpu.VMEM((_BPW, E), jnp.float32),
        pltpu.VMEM((_BPW, E), jnp.float32),
        pltpu.VMEM((_NPW, E), jnp.float32),
        pltpu.VMEM((_NPW, E), jnp.float32),
        pltpu.SemaphoreType.DMA,
        pltpu.SemaphoreType.DMA,
        pltpu.SemaphoreType.DMA,
        pltpu.SemaphoreType.DMA,
    ],
)
def _sc_gather(uids_hbm, iids_hbm, oids_hbm, UI_hbm, LI_hbm, IU_hbm, IL_hbm,
               ue_out, se_out, iu_out, il_out,
               uid_v, iid_v, oid_v, ue_v, se_v, iu_v, il_v,
               sem0, sem1, sem2, sem3):
    wid = lax.axis_index("s") * _NC + lax.axis_index("c")
    bbase = wid * _BPW
    nbase = wid * _NPW

    pltpu.sync_copy(uids_hbm.at[pl.ds(bbase, _BPW)], uid_v)
    pltpu.sync_copy(iids_hbm.at[pl.ds(bbase, _BPW)], iid_v)
    pltpu.sync_copy(oids_hbm.at[pl.ds(nbase, _NPW)], oid_v)

    g0 = pltpu.async_copy(UI_hbm.at[uid_v], ue_v, sem0)
    g1 = pltpu.async_copy(LI_hbm.at[iid_v], se_v, sem1)
    g2 = pltpu.async_copy(IU_hbm.at[oid_v], iu_v, sem2)
    g3 = pltpu.async_copy(IL_hbm.at[oid_v], il_v, sem3)
    g0.wait()
    g1.wait()
    g2.wait()
    g3.wait()

    pltpu.sync_copy(ue_v, ue_out.at[pl.ds(bbase, _BPW)])
    pltpu.sync_copy(se_v, se_out.at[pl.ds(bbase, _BPW)])
    pltpu.sync_copy(iu_v, iu_out.at[pl.ds(nbase, _NPW)])
    pltpu.sync_copy(il_v, il_out.at[pl.ds(nbase, _NPW)])


_BM = 1024


def _mm_body(ue_ref, se_ref, iu_ref, il_ref, out_ref):
    mf = lax.dot_general(ue_ref[...], iu_ref[...], (((1,), (1,)), ((), ())),
                         precision=lax.Precision.HIGHEST,
                         preferred_element_type=jnp.float32)
    fmc = lax.dot_general(se_ref[...], il_ref[...], (((1,), (1,)), ((), ())),
                          precision=lax.Precision.HIGHEST,
                          preferred_element_type=jnp.float32)
    out_ref[...] = mf + fmc


_matmul = pl.pallas_call(
    _mm_body,
    grid=(B // _BM,),
    in_specs=[
        pl.BlockSpec((_BM, E), lambda i: (i, 0)),
        pl.BlockSpec((_BM, E), lambda i: (i, 0)),
        pl.BlockSpec((N, E), lambda i: (0, 0)),
        pl.BlockSpec((N, E), lambda i: (0, 0)),
    ],
    out_specs=pl.BlockSpec((_BM, N), lambda i: (i, 0)),
    out_shape=jax.ShapeDtypeStruct((B, N), jnp.float32),
)


def kernel(in_uids, in_iids, out_iids, UI, IU, LI, IL):
    uids = in_uids.astype(jnp.int32)
    iids = in_iids.astype(jnp.int32)
    oids = out_iids.astype(jnp.int32)
    ue, se, iu, il = _sc_gather(uids, iids, oids, UI, LI, IU, IL)
    return _matmul(ue, se, iu, il)


# speedup vs baseline: 1.5014x; 1.5014x over previous
import functools

import jax
import jax.numpy as jnp
from jax import lax
from jax.experimental import pallas as pl
from jax.experimental.pallas import tpu as pltpu
from jax.experimental.pallas import tpu_sc as plsc

E = 64
B = 4096
N = 1024

_info = plsc.get_sparse_core_info()
_NC, _NS = _info.num_cores, _info.num_subcores
_NW = _NC * _NS
_BPW = B // _NW
_NPW = N // _NW
_CHUNK = 16

_sc_mesh = plsc.VectorSubcoreMesh(core_axis_name="c", subcore_axis_name="s")


def _gather_rows(table_hbm, idx_v, rows_v, sem, n_rows):
    def chunk_body(c, _):
        base = c * _CHUNK
        idxvec = idx_v[pl.ds(base, _CHUNK)]
        copies = []
        for j in range(_CHUNK):
            idx = idxvec[j]
            copies.append(pltpu.async_copy(
                table_hbm.at[pl.ds(idx, 1), :],
                rows_v.at[pl.ds(base + j, 1), :],
                sem))
        for cp in copies:
            cp.wait()
        return 0
    lax.fori_loop(0, n_rows // _CHUNK, chunk_body, 0, unroll=False)


@functools.partial(
    pl.kernel,
    mesh=_sc_mesh,
    out_type=[
        jax.ShapeDtypeStruct((B, E), jnp.float32),
        jax.ShapeDtypeStruct((B, E), jnp.float32),
        jax.ShapeDtypeStruct((N, E), jnp.float32),
        jax.ShapeDtypeStruct((N, E), jnp.float32),
    ],
    scratch_types=[
        pltpu.VMEM((_BPW,), jnp.int32),
        pltpu.VMEM((_BPW,), jnp.int32),
        pltpu.VMEM((_NPW,), jnp.int32),
        pltpu.VMEM((_BPW, E), jnp.float32),
        pltpu.VMEM((_BPW, E), jnp.float32),
        pltpu.VMEM((_NPW, E), jnp.float32),
        pltpu.VMEM((_NPW, E), jnp.float32),
        pltpu.SemaphoreType.DMA,
    ],
)
def _sc_gather(uids_hbm, iids_hbm, oids_hbm, UI_hbm, LI_hbm, IU_hbm, IL_hbm,
               ue_out, se_out, iu_out, il_out,
               uid_s, iid_s, oid_s, ue_v, se_v, iu_v, il_v, sem):
    wid = lax.axis_index("s") * _NC + lax.axis_index("c")
    bbase = wid * _BPW
    nbase = wid * _NPW

    pltpu.sync_copy(uids_hbm.at[pl.ds(bbase, _BPW)], uid_s)
    pltpu.sync_copy(iids_hbm.at[pl.ds(bbase, _BPW)], iid_s)
    pltpu.sync_copy(oids_hbm.at[pl.ds(nbase, _NPW)], oid_s)

    _gather_rows(UI_hbm, uid_s, ue_v, sem, _BPW)
    _gather_rows(LI_hbm, iid_s, se_v, sem, _BPW)
    _gather_rows(IU_hbm, oid_s, iu_v, sem, _NPW)
    _gather_rows(IL_hbm, oid_s, il_v, sem, _NPW)

    pltpu.sync_copy(ue_v, ue_out.at[pl.ds(bbase, _BPW)])
    pltpu.sync_copy(se_v, se_out.at[pl.ds(bbase, _BPW)])
    pltpu.sync_copy(iu_v, iu_out.at[pl.ds(nbase, _NPW)])
    pltpu.sync_copy(il_v, il_out.at[pl.ds(nbase, _NPW)])


_BM = 1024


def _mm_body(ue_ref, se_ref, iu_ref, il_ref, out_ref):
    mf = lax.dot_general(ue_ref[...], iu_ref[...], (((1,), (1,)), ((), ())),
                         precision=lax.Precision.HIGHEST,
                         preferred_element_type=jnp.float32)
    fmc = lax.dot_general(se_ref[...], il_ref[...], (((1,), (1,)), ((), ())),
                          precision=lax.Precision.HIGHEST,
                          preferred_element_type=jnp.float32)
    out_ref[...] = mf + fmc


_matmul = pl.pallas_call(
    _mm_body,
    grid=(B // _BM,),
    in_specs=[
        pl.BlockSpec((_BM, E), lambda i: (i, 0)),
        pl.BlockSpec((_BM, E), lambda i: (i, 0)),
        pl.BlockSpec((N, E), lambda i: (0, 0)),
        pl.BlockSpec((N, E), lambda i: (0, 0)),
    ],
    out_specs=pl.BlockSpec((_BM, N), lambda i: (i, 0)),
    out_shape=jax.ShapeDtypeStruct((B, N), jnp.float32),
)


def kernel(in_uids, in_iids, out_iids, UI, IU, LI, IL):
    uids = in_uids.astype(jnp.int32)
    iids = in_iids.astype(jnp.int32)
    oids = out_iids.astype(jnp.int32)
    ue, se, iu, il = _sc_gather(uids, iids, oids, UI, LI, IU, IL)
    return _matmul(ue, se, iu, il)


# speedup vs baseline: 8.1352x; 5.4183x over previous
import functools

import jax
import jax.numpy as jnp
from jax import lax
from jax.experimental import pallas as pl
from jax.experimental.pallas import tpu as pltpu
from jax.experimental.pallas import tpu_sc as plsc

E = 64
B = 4096
N = 1024
LANES = 128

_info = plsc.get_sparse_core_info()
_NC, _NS = _info.num_cores, _info.num_subcores
_NW = _NC * _NS
_BPW = B // _NW
_NPW = N // _NW
_BURST = 8

_sc_mesh = plsc.VectorSubcoreMesh(core_axis_name="c", subcore_axis_name="s")


def _gather_ids(tableT_hbm, idx_v, rows_v, slab_v, sem, n_ids):
    e16 = lax.iota(jnp.int32, 16)

    def chunk_body(c, _):
        cbase = c * 16
        idxvec = idx_v[pl.ds(cbase, 16)]
        for h in range(16 // _BURST):
            copies = []
            for j in range(_BURST):
                idx = idxvec[h * _BURST + j]
                start = pl.multiple_of((idx >> 7) << 7, LANES)
                copies.append(pltpu.async_copy(
                    tableT_hbm.at[:, pl.ds(start, LANES)],
                    slab_v.at[j],
                    sem))
            for cp in copies:
                cp.wait()
            for j in range(_BURST):
                idx = idxvec[h * _BURST + j]
                lane = jnp.full((16,), idx & 127, jnp.int32)
                pos = cbase + h * _BURST + j
                for k in range(E // 16):
                    vals = plsc.load_gather(slab_v.at[j], [e16 + k * 16, lane])
                    rows_v[pos, pl.ds(k * 16, 16)] = vals
        return 0
    lax.fori_loop(0, n_ids // 16, chunk_body, 0, unroll=False)


@functools.partial(
    pl.kernel,
    mesh=_sc_mesh,
    compiler_params=pltpu.CompilerParams(needs_layout_passes=False),
    out_type=[
        jax.ShapeDtypeStruct((B, E), jnp.float32),
        jax.ShapeDtypeStruct((B, E), jnp.float32),
        jax.ShapeDtypeStruct((N, E), jnp.float32),
        jax.ShapeDtypeStruct((N, E), jnp.float32),
    ],
    scratch_types=[
        pltpu.VMEM((_BPW,), jnp.int32),
        pltpu.VMEM((_BPW,), jnp.int32),
        pltpu.VMEM((_NPW,), jnp.int32),
        pltpu.VMEM((_BPW, E), jnp.float32),
        pltpu.VMEM((_BPW, E), jnp.float32),
        pltpu.VMEM((_NPW, E), jnp.float32),
        pltpu.VMEM((_NPW, E), jnp.float32),
        pltpu.VMEM((_BURST, E, LANES), jnp.float32),
        pltpu.SemaphoreType.DMA,
    ],
)
def _sc_gather(uids_hbm, iids_hbm, oids_hbm, UIt_hbm, LIt_hbm, IUt_hbm, ILt_hbm,
               ue_out, se_out, iu_out, il_out,
               uid_v, iid_v, oid_v, ue_v, se_v, iu_v, il_v, slab_v, sem):
    wid = lax.axis_index("s") * _NC + lax.axis_index("c")
    bbase = wid * _BPW
    nbase = wid * _NPW

    pltpu.sync_copy(uids_hbm.at[pl.ds(bbase, _BPW)], uid_v)
    pltpu.sync_copy(iids_hbm.at[pl.ds(bbase, _BPW)], iid_v)
    pltpu.sync_copy(oids_hbm.at[pl.ds(nbase, _NPW)], oid_v)

    _gather_ids(UIt_hbm, uid_v, ue_v, slab_v, sem, _BPW)
    _gather_ids(LIt_hbm, iid_v, se_v, slab_v, sem, _BPW)
    _gather_ids(IUt_hbm, oid_v, iu_v, slab_v, sem, _NPW)
    _gather_ids(ILt_hbm, oid_v, il_v, slab_v, sem, _NPW)

    pltpu.sync_copy(ue_v, ue_out.at[pl.ds(bbase, _BPW)])
    pltpu.sync_copy(se_v, se_out.at[pl.ds(bbase, _BPW)])
    pltpu.sync_copy(iu_v, iu_out.at[pl.ds(nbase, _NPW)])
    pltpu.sync_copy(il_v, il_out.at[pl.ds(nbase, _NPW)])


_BM = 1024


def _mm_body(ue_ref, se_ref, iu_ref, il_ref, out_ref):
    mf = lax.dot_general(ue_ref[...], iu_ref[...], (((1,), (1,)), ((), ())),
                         precision=lax.Precision.HIGHEST,
                         preferred_element_type=jnp.float32)
    fmc = lax.dot_general(se_ref[...], il_ref[...], (((1,), (1,)), ((), ())),
                          precision=lax.Precision.HIGHEST,
                          preferred_element_type=jnp.float32)
    out_ref[...] = mf + fmc


_matmul = pl.pallas_call(
    _mm_body,
    grid=(B // _BM,),
    in_specs=[
        pl.BlockSpec((_BM, E), lambda i: (i, 0)),
        pl.BlockSpec((_BM, E), lambda i: (i, 0)),
        pl.BlockSpec((N, E), lambda i: (0, 0)),
        pl.BlockSpec((N, E), lambda i: (0, 0)),
    ],
    out_specs=pl.BlockSpec((_BM, N), lambda i: (i, 0)),
    out_shape=jax.ShapeDtypeStruct((B, N), jnp.float32),
)


def kernel(in_uids, in_iids, out_iids, UI, IU, LI, IL):
    uids = in_uids.astype(jnp.int32)
    iids = in_iids.astype(jnp.int32)
    oids = out_iids.astype(jnp.int32)
    ue, se, iu, il = _sc_gather(uids, iids, oids, UI.T, LI.T, IU.T, IL.T)
    return _matmul(ue, se, iu, il)
